# baseline (speedup 1.0000x reference)
import functools

import jax
import jax.numpy as jnp
from jax import lax
from jax.experimental import pallas as pl
from jax.experimental.pallas import tpu as pltpu
from jax.experimental.pallas import tpu_sc as plsc

B = 4
T = 2048
D = 1024
NC = 2
NS = 16
NW = NC * NS
T_PER_W = T // NW
CT = 8
NTC = T_PER_W // CT
ROWS = B * CT
NBUF = 3
LANES = 16

_mesh = plsc.VectorSubcoreMesh(core_axis_name="c", subcore_axis_name="s")


@functools.partial(
    pl.kernel,
    mesh=_mesh,
    out_type=jax.ShapeDtypeStruct((B * T, D), jnp.float32),
    scratch_types=[
        pltpu.VMEM((NTC, ROWS), jnp.int32),
        pltpu.VMEM((CT, D), jnp.float32),
        pltpu.VMEM((CT, D), jnp.float32),
    ]
    + [pltpu.VMEM((ROWS, D), jnp.float32) for _ in range(NBUF)]
    + [pltpu.SemaphoreType.DMA for _ in range(2 + 2 * NBUF + 1)],
)
def _embed(idx_hbm, tok_hbm, pos_hbm, out_hbm, idx_v, *rest):
    posb = rest[0:2]
    bufs = rest[2:2 + NBUF]
    psem = rest[2 + NBUF:4 + NBUF]
    gsem = rest[4 + NBUF:4 + 2 * NBUF]
    ssem = rest[4 + 2 * NBUF:4 + 3 * NBUF]
    isem = rest[4 + 3 * NBUF]
    wid = lax.axis_index("s") * NC + lax.axis_index("c")
    t0 = wid * T_PER_W

    idx_handles = [
        pltpu.async_copy(idx_hbm.at[pl.ds(b * T + t0 + p * CT, CT)],
                         idx_v.at[p].at[pl.ds(b * CT, CT)], isem)
        for p in range(NTC) for b in range(B)
    ]
    for h in idx_handles:
        h.wait()

    def issue_pos(p):
        return pltpu.async_copy(
            pos_hbm.at[pl.ds(t0 + p * CT, CT)], posb[p % 2], psem[p % 2])

    def issue_gather(p):
        return pltpu.async_copy(
            tok_hbm.at[idx_v.at[p]], bufs[p % NBUF], gsem[p % NBUF])

    def issue_stores(p):
        k = p % NBUF
        return [
            pltpu.async_copy(
                bufs[k].at[pl.ds(b * CT, CT)],
                out_hbm.at[pl.ds(b * T + t0 + p * CT, CT)], ssem[k])
            for b in range(B)
        ]

    pos_handles = {0: issue_pos(0)}
    gathers = {p: issue_gather(p) for p in range(NBUF - 1)}
    stores = {}
    for p in range(NTC):
        pos_handles[p].wait()
        if p + 1 < NTC:
            pos_handles[p + 1] = issue_pos(p + 1)
        gathers[p].wait()
        nxt = p + NBUF - 1
        if nxt < NTC:
            if p >= 1:
                for h in stores[p - 1]:
                    h.wait()
            gathers[nxt] = issue_gather(nxt)
        buf, pos = bufs[p % NBUF], posb[p % 2]
        k = p % NBUF
        stores[p] = []
        for b in range(B):

            @plsc.parallel_loop(0, CT, 1)
            def _add_rows(r, buf=buf, pos=pos, b=b):
                @plsc.parallel_loop(0, D, LANES, unroll=4)
                def _add_lanes(c, r=r, buf=buf, pos=pos, b=b):
                    sl = pl.ds(c, LANES)
                    plsc.addupdate(buf.at[b * CT + r, sl], pos[r, sl])

            stores[p].append(pltpu.async_copy(
                bufs[k].at[pl.ds(b * CT, CT)],
                out_hbm.at[pl.ds(b * T + t0 + p * CT, CT)], ssem[k]))
    for p in range(max(0, NTC - NBUF), NTC):
        for h in stores[p]:
            h.wait()


def kernel(idx, token_embedding_table, position_embedding_table):
    idx_flat = idx.astype(jnp.int32).reshape(B * T)
    out = _embed(idx_flat, token_embedding_table, position_embedding_table)
    return out.reshape(B, T, D)


# speedup vs baseline: 1.0531x; 1.0531x over previous
import functools

import jax
import jax.numpy as jnp
from jax import lax
from jax.experimental import pallas as pl
from jax.experimental.pallas import tpu as pltpu
from jax.experimental.pallas import tpu_sc as plsc

B = 4
T = 2048
D = 1024
NC = 2
NS = 16
NW = NC * NS
T_PER_W = T // NW
CT = 8
NTC = T_PER_W // CT
DEPTH = 3
LANES = 16

_mesh = plsc.VectorSubcoreMesh(core_axis_name="c", subcore_axis_name="s")


@functools.partial(
    pl.kernel,
    mesh=_mesh,
    out_type=jax.ShapeDtypeStruct((B * T, D), jnp.float32),
    scratch_types=[
        pltpu.VMEM((B, T_PER_W), jnp.int32),
        pltpu.VMEM((CT, D), jnp.float32),
        pltpu.VMEM((CT, D), jnp.float32),
    ]
    + [pltpu.VMEM((CT, D), jnp.float32) for _ in range(DEPTH * B)]
    + [pltpu.SemaphoreType.DMA for _ in range(2 + 2 * DEPTH * B + 1)],
)
def _embed(idx_hbm, tok_hbm, pos_hbm, out_hbm, idx_v, *rest):
    nb = DEPTH * B
    posb = rest[0:2]
    toks = rest[2:2 + nb]
    psem = rest[2 + nb:4 + nb]
    gsem = rest[4 + nb:4 + 2 * nb]
    ssem = rest[4 + 2 * nb:4 + 3 * nb]
    isem = rest[4 + 3 * nb]
    wid = lax.axis_index("s") * NC + lax.axis_index("c")
    t0 = wid * T_PER_W

    idx_handles = [
        pltpu.async_copy(idx_hbm.at[pl.ds(b * T + t0, T_PER_W)],
                         idx_v.at[b], isem)
        for b in range(B)
    ]
    for h in idx_handles:
        h.wait()

    def issue_pos(p):
        return pltpu.async_copy(
            pos_hbm.at[pl.ds(t0 + p * CT, CT)], posb[p % 2], psem[p % 2])

    def issue_gather(p, b):
        k = (p % DEPTH) * B + b
        return pltpu.async_copy(
            tok_hbm.at[idx_v.at[b].at[pl.ds(p * CT, CT)]], toks[k], gsem[k])

    pos_handles = {0: issue_pos(0)}
    gathers = {(p, b): issue_gather(p, b)
               for p in range(DEPTH - 1) for b in range(B)}
    stores = {}
    for p in range(NTC):
        s = (p % DEPTH) * B
        pos_handles[p].wait()
        if p + 1 < NTC:
            pos_handles[p + 1] = issue_pos(p + 1)
        for b in range(B):
            gathers[(p, b)].wait()
        nxt = p + DEPTH - 1
        if nxt < NTC:
            for b in range(B):
                if nxt - DEPTH >= 0:
                    stores[(nxt - DEPTH, b)].wait()
                gathers[(nxt, b)] = issue_gather(nxt, b)
        t_s, pos = toks[s:s + B], posb[p % 2]

        @plsc.parallel_loop(0, CT, 1)
        def _add_rows(r, t_s=t_s, pos=pos):
            @plsc.parallel_loop(0, D, LANES, unroll=4)
            def _add_lanes(c, r=r, t_s=t_s, pos=pos):
                sl = pl.ds(c, LANES)
                pv = pos[r, sl]
                for b in range(B):
                    plsc.addupdate(t_s[b].at[r, sl], pv)

        for b in range(B):
            stores[(p, b)] = pltpu.async_copy(
                toks[s + b], out_hbm.at[pl.ds(b * T + t0 + p * CT, CT)],
                ssem[s + b])
    for p in range(max(0, NTC - DEPTH), NTC):
        for b in range(B):
            stores[(p, b)].wait()


def kernel(idx, token_embedding_table, position_embedding_table):
    idx_flat = idx.astype(jnp.int32).reshape(B * T)
    out = _embed(idx_flat, token_embedding_table, position_embedding_table)
    return out.reshape(B, T, D)


# speedup vs baseline: 1.0646x; 1.0109x over previous
import functools

import jax
import jax.numpy as jnp
from jax import lax
from jax.experimental import pallas as pl
from jax.experimental.pallas import tpu as pltpu
from jax.experimental.pallas import tpu_sc as plsc

B = 4
T = 2048
D = 1024
NC = 2
NS = 16
NW = NC * NS
T_PER_W = T // NW
CT = 8
NTC = T_PER_W // CT
DEPTH = 3
LANES = 16

_mesh = plsc.VectorSubcoreMesh(core_axis_name="c", subcore_axis_name="s")


@functools.partial(
    pl.kernel,
    mesh=_mesh,
    out_type=jax.ShapeDtypeStruct((B * T, D), jnp.float32),
    scratch_types=[
        pltpu.VMEM((B, T_PER_W), jnp.int32),
        pltpu.VMEM((CT, D), jnp.float32),
        pltpu.VMEM((CT, D), jnp.float32),
    ]
    + [pltpu.VMEM((CT, D), jnp.float32) for _ in range(DEPTH * B)]
    + [pltpu.SemaphoreType.DMA for _ in range(2 + 2 * DEPTH * B + 1)],
)
def _embed(idx_hbm, tok_hbm, pos_hbm, out_hbm, idx_v, *rest):
    nb = DEPTH * B
    posb = rest[0:2]
    toks = rest[2:2 + nb]
    psem = rest[2 + nb:4 + nb]
    gsem = rest[4 + nb:4 + 2 * nb]
    ssem = rest[4 + 2 * nb:4 + 3 * nb]
    isem = rest[4 + 3 * nb]
    wid = lax.axis_index("s") * NC + lax.axis_index("c")
    t0 = wid * T_PER_W

    idx_handles = [
        pltpu.async_copy(idx_hbm.at[pl.ds(b * T + t0, T_PER_W)],
                         idx_v.at[b], isem)
        for b in range(B)
    ]
    for h in idx_handles:
        h.wait()

    def issue_pos(p):
        return pltpu.async_copy(
            pos_hbm.at[pl.ds(t0 + p * CT, CT)], posb[p % 2], psem[p % 2])

    def issue_gather(p, b):
        k = (p % DEPTH) * B + b
        return pltpu.async_copy(
            tok_hbm.at[idx_v.at[b].at[pl.ds(p * CT, CT)]], toks[k], gsem[k])

    pos_handles = {0: issue_pos(0)}
    gathers = {(p, b): issue_gather(p, b)
               for p in range(DEPTH - 1) for b in range(B)}
    stores = {}
    for p in range(NTC):
        s = (p % DEPTH) * B
        pos_handles[p].wait()
        if p + 1 < NTC:
            pos_handles[p + 1] = issue_pos(p + 1)
        for b in range(B):
            gathers[(p, b)].wait()
        nxt = p + DEPTH - 1
        if nxt < NTC:
            for b in range(B):
                if nxt - DEPTH >= 0:
                    stores[(nxt - DEPTH, b)].wait()
                gathers[(nxt, b)] = issue_gather(nxt, b)
        t_s, pos = toks[s:s + B], posb[p % 2]

        @plsc.parallel_loop(0, CT, 1)
        def _add_rows(r, t_s=t_s, pos=pos):
            @plsc.parallel_loop(0, D, LANES, unroll=4)
            def _add_lanes(c, r=r, t_s=t_s, pos=pos):
                sl = pl.ds(c, LANES)
                pv = pos[r, sl]
                for b in range(B):
                    t_s[b][r, sl] += pv

        for b in range(B):
            stores[(p, b)] = pltpu.async_copy(
                toks[s + b], out_hbm.at[pl.ds(b * T + t0 + p * CT, CT)],
                ssem[s + b])
    for p in range(max(0, NTC - DEPTH), NTC):
        for b in range(B):
            stores[(p, b)].wait()


def kernel(idx, token_embedding_table, position_embedding_table):
    idx_flat = idx.astype(jnp.int32).reshape(B * T)
    out = _embed(idx_flat, token_embedding_table, position_embedding_table)
    return out.reshape(B, T, D)
